# Initial kernel scaffold; baseline (speedup 1.0000x reference)
import jax
import jax.numpy as jnp
from jax.experimental import pallas as pl


def kernel(x, pos_table):
    raise NotImplementedError("write your pallas kernel here")


# speedup vs baseline: 1.4986x; 1.4986x over previous
import jax
import jax.numpy as jnp
from jax.experimental import pallas as pl

BLOCK_S = 512


def _add_kernel(x_ref, pos_ref, out_ref):
    out_ref[...] = x_ref[...] + pos_ref[...][None, :, :]


def kernel(x, pos_table):
    batch, seq_len, d_model = x.shape
    num_s = seq_len // BLOCK_S
    return pl.pallas_call(
        _add_kernel,
        grid=(num_s, batch),
        in_specs=[
            pl.BlockSpec((1, BLOCK_S, d_model), lambda s, b: (b, s, 0)),
            pl.BlockSpec((BLOCK_S, d_model), lambda s, b: (s, 0)),
        ],
        out_specs=pl.BlockSpec((1, BLOCK_S, d_model), lambda s, b: (b, s, 0)),
        out_shape=jax.ShapeDtypeStruct(x.shape, x.dtype),
    )(x, pos_table[:seq_len])
